# baseline (speedup 1.0000x reference)
import functools

import jax
import jax.numpy as jnp
from jax import lax
from jax.experimental import pallas as pl
from jax.experimental.pallas import tpu as pltpu
from jax.experimental.pallas import tpu_sc as plsc

N_NODES = 10000
N_EDGES = 320000
D = 128
DH = D // 2

NC = 2
NS = 16
EPW = N_EDGES // NS
K = 80
NCHUNK = EPW // K
NBUF = 5
N_PAD = 10240
RPS = N_PAD // NS
ZROWS = 128
CW = 16


def _sc_body(compute_cnt, xlo_hbm, xhi_hbm, src2_hbm, dst2_hbm, agg_out,
             cnt_out, srcs, dsts, rows0, rows1, rows2, rows3, rows4, ones_v,
             zbuf, zcnt, agg_sh, cnt_sh, sem0, sem1, sem2, sem3, sem4, csem):
    cid = lax.axis_index("c")
    sid = lax.axis_index("s")

    coff = pl.multiple_of(sid * NCHUNK, NCHUNK)
    pltpu.sync_copy(src2_hbm.at[pl.ds(coff, NCHUNK)], srcs)
    pltpu.sync_copy(dst2_hbm.at[pl.ds(coff, NCHUNK)], dsts)

    z16 = jnp.zeros((16,), jnp.float32)
    o16 = jnp.ones((16,), jnp.float32)

    def zb(i, c):
        for j in range(DH // 16):
            zbuf[i, pl.ds(j * 16, 16)] = z16
        zcnt[i, :] = z16
        return c
    lax.fori_loop(0, ZROWS, zb, 0)

    if compute_cnt:
        def ob(i, c):
            ones_v[i, :] = o16
            return c
        lax.fori_loop(0, K, ob, 0)

    for t in range(RPS // ZROWS):
        off = pl.multiple_of(sid * RPS + t * ZROWS, ZROWS)
        pltpu.sync_copy(zbuf, agg_sh.at[pl.ds(off, ZROWS)])
        if compute_cnt:
            pltpu.sync_copy(zcnt, cnt_sh.at[pl.ds(off, ZROWS)])
    plsc.subcore_barrier()

    def fire(c, buf, sem):
        @pl.when(cid == 0)
        def _():
            pltpu.async_copy(xlo_hbm.at[srcs.at[c]], buf, sem)

        @pl.when(cid == 1)
        def _():
            pltpu.async_copy(xhi_hbm.at[srcs.at[c]], buf, sem)

    def drain(buf, sem):
        pltpu.make_async_copy(xlo_hbm.at[srcs.at[0]], buf, sem).wait()

    def consume(c, buf):
        pltpu.sync_copy(buf, agg_sh.at[dsts.at[c]], add=True)
        if compute_cnt:
            @pl.when(jnp.logical_xor(cid == 1, c < NCHUNK // 2))
            def _():
                pltpu.async_copy(ones_v, cnt_sh.at[dsts.at[c]], csem,
                                 add=True)

    bufs = (rows0, rows1, rows2, rows3, rows4)
    sems = (sem0, sem1, sem2, sem3, sem4)
    for b in range(NBUF):
        fire(b, bufs[b], sems[b])

    def body(i, carry):
        c0 = i * NBUF
        for b in range(NBUF):
            drain(bufs[b], sems[b])
            consume(c0 + b, bufs[b])

            @pl.when(c0 + b + NBUF < NCHUNK)
            def _():
                fire(c0 + b + NBUF, bufs[b], sems[b])
        return carry
    lax.fori_loop(0, NCHUNK // NBUF, body, 0)
    if compute_cnt:
        def cdrain(i, carry):
            pltpu.make_async_copy(ones_v, cnt_sh.at[dsts.at[0]],
                                  csem).wait()
            return carry
        lax.fori_loop(0, NCHUNK // 2, cdrain, 0)
    plsc.subcore_barrier()

    roff = pl.multiple_of(sid * RPS, RPS)
    pltpu.sync_copy(agg_sh.at[pl.ds(roff, RPS)],
                    agg_out.at[cid, pl.ds(roff, RPS)])
    if compute_cnt:
        pltpu.sync_copy(cnt_sh.at[pl.ds(roff, RPS)],
                        cnt_out.at[cid, pl.ds(roff, RPS)])


def _make_sc_kernel(compute_cnt):
    mesh = plsc.VectorSubcoreMesh(core_axis_name="c", subcore_axis_name="s")
    out_type = [jax.ShapeDtypeStruct((NC, N_PAD, DH), jnp.float32)]
    if compute_cnt:
        out_type.append(jax.ShapeDtypeStruct((NC, N_PAD, CW), jnp.float32))
    scratch = [
        pltpu.VMEM((NCHUNK, K), jnp.int32),
        pltpu.VMEM((NCHUNK, K), jnp.int32),
        pltpu.VMEM((K, DH), jnp.float32),
        pltpu.VMEM((K, DH), jnp.float32),
        pltpu.VMEM((K, DH), jnp.float32),
        pltpu.VMEM((K, DH), jnp.float32),
        pltpu.VMEM((K, DH), jnp.float32),
        pltpu.VMEM((K, CW), jnp.float32),
        pltpu.VMEM((ZROWS, DH), jnp.float32),
        pltpu.VMEM((ZROWS, CW), jnp.float32),
        pltpu.VMEM_SHARED((N_PAD, DH), jnp.float32),
        pltpu.VMEM_SHARED((N_PAD, CW), jnp.float32),
        pltpu.SemaphoreType.DMA,
        pltpu.SemaphoreType.DMA,
        pltpu.SemaphoreType.DMA,
        pltpu.SemaphoreType.DMA,
        pltpu.SemaphoreType.DMA,
        pltpu.SemaphoreType.DMA,
    ]
    if compute_cnt:
        body = functools.partial(_sc_body, True)
    else:
        def body(xlo_hbm, xhi_hbm, src2_hbm, dst2_hbm, agg_out, *rest):
            return _sc_body(False, xlo_hbm, xhi_hbm, src2_hbm, dst2_hbm,
                            agg_out, None, *rest)
    return pl.kernel(
        body,
        out_type=out_type,
        mesh=mesh,
        scratch_types=scratch,
        compiler_params=pltpu.CompilerParams(use_tc_tiling_on_sc=False),
    )


def _tc_body(relu, split_out, agg_ref, cnt_ref, xlo_ref, xhi_ref, wl_ref,
             wr_ref, b_ref, *out_refs):
    cnt = jnp.maximum(cnt_ref[0, :, :1] + cnt_ref[1, :, :1], 1.0)
    mean_lo = agg_ref[0] / cnt
    mean_hi = agg_ref[1] / cnt
    wl = wl_ref[...]
    wr = wr_ref[...]
    acc = jnp.dot(mean_lo, wl[:DH, :], preferred_element_type=jnp.float32)
    acc = acc + jnp.dot(mean_hi, wl[DH:, :],
                        preferred_element_type=jnp.float32)
    acc = acc + jnp.dot(xlo_ref[...], wr[:DH, :],
                        preferred_element_type=jnp.float32)
    acc = acc + jnp.dot(xhi_ref[...], wr[DH:, :],
                        preferred_element_type=jnp.float32)
    acc = acc + b_ref[...]
    if relu:
        acc = jnp.maximum(acc, 0.0)
    if split_out:
        out_refs[0][...] = acc[:, :DH]
        out_refs[1][...] = acc[:, DH:]
    else:
        out_refs[0][...] = acc


BN = 5000


def _make_tc_kernel(relu, split_out):
    grid = (N_NODES // BN,)
    if split_out:
        out_specs = [pl.BlockSpec((BN, DH), lambda i: (i, 0)),
                     pl.BlockSpec((BN, DH), lambda i: (i, 0))]
        out_shape = [jax.ShapeDtypeStruct((N_NODES, DH), jnp.float32),
                     jax.ShapeDtypeStruct((N_NODES, DH), jnp.float32)]
    else:
        out_specs = pl.BlockSpec((BN, D), lambda i: (i, 0))
        out_shape = jax.ShapeDtypeStruct((N_NODES, D), jnp.float32)
    return pl.pallas_call(
        functools.partial(_tc_body, relu, split_out),
        grid=grid,
        in_specs=[
            pl.BlockSpec((NC, BN, DH), lambda i: (0, i, 0)),
            pl.BlockSpec((NC, BN, CW), lambda i: (0, i, 0)),
            pl.BlockSpec((BN, DH), lambda i: (i, 0)),
            pl.BlockSpec((BN, DH), lambda i: (i, 0)),
            pl.BlockSpec((D, D), lambda i: (0, 0)),
            pl.BlockSpec((D, D), lambda i: (0, 0)),
            pl.BlockSpec((1, D), lambda i: (0, 0)),
        ],
        out_specs=out_specs,
        out_shape=out_shape,
    )


_sc_agg_cnt = _make_sc_kernel(True)
_sc_agg = _make_sc_kernel(False)
_tc_layer1 = _make_tc_kernel(True, True)
_tc_layer2 = _make_tc_kernel(False, False)


def kernel(x, edge_index, W1_l, W1_r, b1, W2_l, W2_r, b2):
    src2 = edge_index[0].reshape(N_EDGES // K, K)
    dst2 = edge_index[1].reshape(N_EDGES // K, K)
    x_lo = x[:, :DH]
    x_hi = x[:, DH:]
    agg1, cnt = _sc_agg_cnt(x_lo, x_hi, src2, dst2)
    h_lo, h_hi = _tc_layer1(agg1, cnt, x_lo, x_hi, W1_l, W1_r,
                            b1.reshape(1, D))
    (agg2,) = _sc_agg(h_lo, h_hi, src2, dst2)
    out = _tc_layer2(agg2, cnt, h_lo, h_hi, W2_l, W2_r, b2.reshape(1, D))
    return out


# speedup vs baseline: 1.0043x; 1.0043x over previous
import functools

import jax
import jax.numpy as jnp
from jax import lax
from jax.experimental import pallas as pl
from jax.experimental.pallas import tpu as pltpu
from jax.experimental.pallas import tpu_sc as plsc

N_NODES = 10000
N_EDGES = 320000
D = 128
DH = D // 2

NC = 2
NS = 16
EPW = N_EDGES // NS
K = 80
NCHUNK = EPW // K
NBUF = 5
N_PAD = 10240
RPS = N_PAD // NS
ZROWS = 128
CW = 16


def _sc_body(compute_cnt, xlo_hbm, xhi_hbm, src2_hbm, dst2_hbm, agg_out,
             cnt_out, srcs, dsts, rows0, rows1, rows2, rows3, rows4, ones_v,
             zbuf, zcnt, agg_sh, cnt_sh, sem0, sem1, sem2, sem3, sem4, csem):
    cid = lax.axis_index("c")
    sid = lax.axis_index("s")

    coff = pl.multiple_of(sid * NCHUNK, NCHUNK)
    pltpu.sync_copy(src2_hbm.at[pl.ds(coff, NCHUNK)], srcs)
    pltpu.sync_copy(dst2_hbm.at[pl.ds(coff, NCHUNK)], dsts)

    z16 = jnp.zeros((16,), jnp.float32)
    o16 = jnp.ones((16,), jnp.float32)

    def zb(i, c):
        for j in range(DH // 16):
            zbuf[i, pl.ds(j * 16, 16)] = z16
        zcnt[i, :] = z16
        return c
    lax.fori_loop(0, ZROWS, zb, 0)

    if compute_cnt:
        def ob(i, c):
            ones_v[i, :] = o16
            return c
        lax.fori_loop(0, K, ob, 0)

    for t in range(RPS // ZROWS):
        off = pl.multiple_of(sid * RPS + t * ZROWS, ZROWS)
        pltpu.sync_copy(zbuf, agg_sh.at[pl.ds(off, ZROWS)])
        if compute_cnt:
            pltpu.sync_copy(zcnt, cnt_sh.at[pl.ds(off, ZROWS)])
    plsc.subcore_barrier()

    def fire(c, buf, sem):
        @pl.when(cid == 0)
        def _():
            pltpu.async_copy(xlo_hbm.at[srcs.at[c]], buf, sem)

        @pl.when(cid == 1)
        def _():
            pltpu.async_copy(xhi_hbm.at[srcs.at[c]], buf, sem)

    def drain(buf, sem):
        pltpu.make_async_copy(xlo_hbm.at[srcs.at[0]], buf, sem).wait()

    def consume(c, buf):
        pltpu.sync_copy(buf, agg_sh.at[dsts.at[c]], add=True)
        if compute_cnt:
            @pl.when(jnp.logical_xor(cid == 1, c < NCHUNK // 2))
            def _():
                pltpu.async_copy(ones_v, cnt_sh.at[dsts.at[c]], csem,
                                 add=True)

    bufs = (rows0, rows1, rows2, rows3, rows4)
    sems = (sem0, sem1, sem2, sem3, sem4)
    for b in range(NBUF):
        fire(b, bufs[b], sems[b])

    def body(i, carry):
        c0 = i * NBUF
        for b in range(NBUF):
            drain(bufs[b], sems[b])
            consume(c0 + b, bufs[b])

            @pl.when(c0 + b + NBUF < NCHUNK)
            def _():
                fire(c0 + b + NBUF, bufs[b], sems[b])
        return carry
    lax.fori_loop(0, NCHUNK // NBUF, body, 0)
    if compute_cnt:
        def cdrain(i, carry):
            pltpu.make_async_copy(ones_v, cnt_sh.at[dsts.at[0]],
                                  csem).wait()
            return carry
        lax.fori_loop(0, NCHUNK // 2, cdrain, 0)
    plsc.subcore_barrier()

    roff = pl.multiple_of(sid * RPS, RPS)
    pltpu.sync_copy(agg_sh.at[pl.ds(roff, RPS)],
                    agg_out.at[cid, pl.ds(roff, RPS)])
    if compute_cnt:
        pltpu.sync_copy(cnt_sh.at[pl.ds(roff, RPS)],
                        cnt_out.at[cid, pl.ds(roff, RPS)])


def _make_sc_kernel(compute_cnt):
    mesh = plsc.VectorSubcoreMesh(core_axis_name="c", subcore_axis_name="s")
    out_type = [jax.ShapeDtypeStruct((NC, N_PAD, DH), jnp.float32)]
    if compute_cnt:
        out_type.append(jax.ShapeDtypeStruct((NC, N_PAD, CW), jnp.float32))
    scratch = [
        pltpu.VMEM((NCHUNK, K), jnp.int32),
        pltpu.VMEM((NCHUNK, K), jnp.int32),
        pltpu.VMEM((K, DH), jnp.float32),
        pltpu.VMEM((K, DH), jnp.float32),
        pltpu.VMEM((K, DH), jnp.float32),
        pltpu.VMEM((K, DH), jnp.float32),
        pltpu.VMEM((K, DH), jnp.float32),
        pltpu.VMEM((K, CW), jnp.float32),
        pltpu.VMEM((ZROWS, DH), jnp.float32),
        pltpu.VMEM((ZROWS, CW), jnp.float32),
        pltpu.VMEM_SHARED((N_PAD, DH), jnp.float32),
        pltpu.VMEM_SHARED((N_PAD, CW), jnp.float32),
        pltpu.SemaphoreType.DMA,
        pltpu.SemaphoreType.DMA,
        pltpu.SemaphoreType.DMA,
        pltpu.SemaphoreType.DMA,
        pltpu.SemaphoreType.DMA,
        pltpu.SemaphoreType.DMA,
    ]
    if compute_cnt:
        body = functools.partial(_sc_body, True)
    else:
        def body(xlo_hbm, xhi_hbm, src2_hbm, dst2_hbm, agg_out, *rest):
            return _sc_body(False, xlo_hbm, xhi_hbm, src2_hbm, dst2_hbm,
                            agg_out, None, *rest)
    return pl.kernel(
        body,
        out_type=out_type,
        mesh=mesh,
        scratch_types=scratch,
        compiler_params=pltpu.CompilerParams(use_tc_tiling_on_sc=False),
    )


def _tc_body(relu, split_out, agg_ref, cnt_ref, xlo_ref, xhi_ref, wl_ref,
             wr_ref, b_ref, *out_refs):
    cnt = jnp.maximum(cnt_ref[0, :, :1] + cnt_ref[1, :, :1], 1.0)
    mean_lo = agg_ref[0] / cnt
    mean_hi = agg_ref[1] / cnt
    wl = wl_ref[...]
    wr = wr_ref[...]
    acc = jnp.dot(mean_lo, wl[:DH, :], preferred_element_type=jnp.float32)
    acc = acc + jnp.dot(mean_hi, wl[DH:, :],
                        preferred_element_type=jnp.float32)
    acc = acc + jnp.dot(xlo_ref[...], wr[:DH, :],
                        preferred_element_type=jnp.float32)
    acc = acc + jnp.dot(xhi_ref[...], wr[DH:, :],
                        preferred_element_type=jnp.float32)
    acc = acc + b_ref[...]
    if relu:
        acc = jnp.maximum(acc, 0.0)
    if split_out:
        out_refs[0][...] = acc[:, :DH]
        out_refs[1][...] = acc[:, DH:]
    else:
        out_refs[0][...] = acc


BN = 2000


def _make_tc_kernel(relu, split_out):
    grid = (N_NODES // BN,)
    if split_out:
        out_specs = [pl.BlockSpec((BN, DH), lambda i: (i, 0)),
                     pl.BlockSpec((BN, DH), lambda i: (i, 0))]
        out_shape = [jax.ShapeDtypeStruct((N_NODES, DH), jnp.float32),
                     jax.ShapeDtypeStruct((N_NODES, DH), jnp.float32)]
    else:
        out_specs = pl.BlockSpec((BN, D), lambda i: (i, 0))
        out_shape = jax.ShapeDtypeStruct((N_NODES, D), jnp.float32)
    return pl.pallas_call(
        functools.partial(_tc_body, relu, split_out),
        grid=grid,
        in_specs=[
            pl.BlockSpec((NC, BN, DH), lambda i: (0, i, 0)),
            pl.BlockSpec((NC, BN, CW), lambda i: (0, i, 0)),
            pl.BlockSpec((BN, DH), lambda i: (i, 0)),
            pl.BlockSpec((BN, DH), lambda i: (i, 0)),
            pl.BlockSpec((D, D), lambda i: (0, 0)),
            pl.BlockSpec((D, D), lambda i: (0, 0)),
            pl.BlockSpec((1, D), lambda i: (0, 0)),
        ],
        out_specs=out_specs,
        out_shape=out_shape,
    )


_sc_agg_cnt = _make_sc_kernel(True)
_sc_agg = _make_sc_kernel(False)
_tc_layer1 = _make_tc_kernel(True, True)
_tc_layer2 = _make_tc_kernel(False, False)


def kernel(x, edge_index, W1_l, W1_r, b1, W2_l, W2_r, b2):
    src2 = edge_index[0].reshape(N_EDGES // K, K)
    dst2 = edge_index[1].reshape(N_EDGES // K, K)
    x_lo = x[:, :DH]
    x_hi = x[:, DH:]
    agg1, cnt = _sc_agg_cnt(x_lo, x_hi, src2, dst2)
    h_lo, h_hi = _tc_layer1(agg1, cnt, x_lo, x_hi, W1_l, W1_r,
                            b1.reshape(1, D))
    (agg2,) = _sc_agg(h_lo, h_hi, src2, dst2)
    out = _tc_layer2(agg2, cnt, h_lo, h_hi, W2_l, W2_r, b2.reshape(1, D))
    return out
